# Initial kernel scaffold; baseline (speedup 1.0000x reference)
import jax
import jax.numpy as jnp
from jax.experimental import pallas as pl


def kernel(fillers, roles, filler_table, role_table, W, b):
    raise NotImplementedError("write your pallas kernel here")


# speedup vs baseline: 1.3838x; 1.3838x over previous
import functools

import jax
import jax.numpy as jnp
from jax import lax
from jax.experimental import pallas as pl
from jax.experimental.pallas import tpu as pltpu
from jax.experimental.pallas import tpu_sc as plsc

NUM_FILLERS = 100000
NUM_ROLES = 64
FILLER_DIM = 64
ROLE_DIM = 32
FINAL_WIDTH = 512
B = 4096
L = 50
LP = 64

NC = 2
NS = 16
NW = NC * NS
BW = B // NW
SR = FILLER_DIM * NUM_ROLES


def _sc_segment_sum(fillers_flat, roles_flat, filler_table):
    mesh = plsc.VectorSubcoreMesh(core_axis_name="c", subcore_axis_name="s")
    LW = BW * LP

    @functools.partial(
        pl.kernel,
        out_type=jax.ShapeDtypeStruct((B, SR), jnp.float32),
        mesh=mesh,
        compiler_params=pltpu.CompilerParams(needs_layout_passes=False,
                                             use_tc_tiling_on_sc=False),
        scratch_types=[
            pltpu.VMEM((LW,), jnp.int32),
            pltpu.VMEM((LW,), jnp.int32),
            pltpu.VMEM((LP, FILLER_DIM), jnp.float32),
            pltpu.VMEM((LP, FILLER_DIM), jnp.float32),
            pltpu.VMEM((SR,), jnp.float32),
            pltpu.SemaphoreType.DMA,
            pltpu.SemaphoreType.DMA,
            pltpu.SemaphoreType.DMA,
            pltpu.SemaphoreType.DMA,
        ],
    )
    def sc_kernel(fillers_hbm, roles_hbm, table_hbm, out_hbm,
                  idx_all, rol_all, rows0, rows1, s_v,
                  sem_ia, sem_ra, sem_g0, sem_g1):
        rows_v = (rows0, rows1)
        sem_g = (sem_g0, sem_g1)
        wid = lax.axis_index("c") * NS + lax.axis_index("s")
        base = wid * BW

        iota = lax.iota(jnp.int32, 16)
        iota64 = iota * 64
        zf = jnp.zeros((16,), jnp.float32)

        cp_i = pltpu.async_copy(fillers_hbm.at[pl.ds(base * LP, LW)], idx_all, sem_ia)
        cp_r = pltpu.async_copy(roles_hbm.at[pl.ds(base * LP, LW)], rol_all, sem_ra)
        cp_i.wait()
        cp_r.wait()
        pltpu.async_copy(table_hbm.at[idx_all.at[pl.ds(0, LP)]], rows_v[0], sem_g[0])

        def body(i, carry):
            for k in range(2):
                cur, nb = k, 1 - k
                ib = 2 * i + k
                jb = jnp.minimum(ib + 1, BW - 1)
                pltpu.make_async_copy(table_hbm.at[idx_all.at[pl.ds(0, LP)]],
                                      rows_v[cur], sem_g[cur]).wait()
                pltpu.async_copy(table_hbm.at[idx_all.at[pl.ds(jb * LP, LP)]],
                                 rows_v[nb], sem_g[nb])
                for t in range(SR // 16):
                    s_v[pl.ds(t * 16, 16)] = zf
                rvecs = [rol_all[pl.ds(ib * LP + g * 16, 16)] for g in range(4)]
                for l in range(L):
                    idx_base = iota64 + rvecs[l // 16][l % 16]
                    for j in range(FILLER_DIM // 16):
                        chunk = rows_v[cur][l, pl.ds(j * 16, 16)]
                        plsc.addupdate_scatter(s_v, [idx_base + (j * 1024)], chunk)
                pltpu.sync_copy(s_v, out_hbm.at[base + ib])
            return carry

        lax.fori_loop(0, BW // 2, body, 0)
        pltpu.make_async_copy(table_hbm.at[idx_all.at[pl.ds(0, LP)]],
                              rows_v[0], sem_g[0]).wait()

    return sc_kernel(fillers_flat, roles_flat, filler_table)


def _tc_contract(s_flat, k8, w, bias2):
    grid = (B // 256,)

    def tc_body(s_ref, k8_ref, w_ref, bias_ref, out_ref, a2_scr):
        @pl.when(pl.program_id(0) == 0)
        def _():
            for a in range(8):
                a2_scr[pl.ds(a * 512, 512), :] = jnp.dot(
                    k8_ref[...], w_ref[pl.ds(a * 256, 256), :],
                    preferred_element_type=jnp.float32)
        out_ref[...] = jnp.dot(s_ref[...], a2_scr[...],
                               preferred_element_type=jnp.float32) + bias_ref[...]

    return pl.pallas_call(
        tc_body,
        grid=grid,
        in_specs=[
            pl.BlockSpec((256, SR), lambda i: (i, 0)),
            pl.BlockSpec((512, 256), lambda i: (0, 0)),
            pl.BlockSpec((FILLER_DIM * ROLE_DIM, FINAL_WIDTH), lambda i: (0, 0)),
            pl.BlockSpec((1, FINAL_WIDTH), lambda i: (0, 0)),
        ],
        out_specs=pl.BlockSpec((256, FINAL_WIDTH), lambda i: (i, 0)),
        out_shape=jax.ShapeDtypeStruct((B, FINAL_WIDTH), jnp.float32),
        scratch_shapes=[pltpu.VMEM((SR, FINAL_WIDTH), jnp.float32)],
    )(s_flat, k8, w, bias2)


def kernel(fillers, roles, filler_table, role_table, W, b):
    pad = ((0, 0), (0, LP - L))
    fillers = jnp.pad(fillers.astype(jnp.int32), pad)
    roles = jnp.pad(roles.astype(jnp.int32), pad)
    s_flat = _sc_segment_sum(fillers.reshape(-1), roles.reshape(-1), filler_table)
    k8 = jnp.kron(jnp.eye(8, dtype=jnp.float32), role_table)
    return _tc_contract(s_flat, k8, W, b.reshape(1, FINAL_WIDTH))
